# Initial kernel scaffold; baseline (speedup 1.0000x reference)
import jax
import jax.numpy as jnp
from jax.experimental import pallas as pl


def kernel(x, adj_row, adj_col, adj_values, kernel):
    raise NotImplementedError("write your pallas kernel here")


# speedup vs baseline: 2.8562x; 2.8562x over previous
import functools

import jax
import jax.numpy as jnp
from jax import lax
from jax.experimental import pallas as pl
from jax.experimental.pallas import tpu as pltpu
from jax.experimental.pallas import tpu_sc as plsc

N_NODES = 10000
N_EDGES = 320000
D_FEAT = 128
OUT_DIM = 256

NC = 2
NS = 16
DHALF = D_FEAT // NC
E_PER_S = N_EDGES // NS
CHUNK = 80
NCHUNK = E_PER_S // CHUNK
ROWS_PER_TILE = N_NODES // NS
LANES = 16


def _sc_aggregate(x2, col3, row3, val3, zeros):
  mesh = plsc.VectorSubcoreMesh(core_axis_name="c", subcore_axis_name="s")

  @functools.partial(
      pl.kernel,
      out_type=jax.ShapeDtypeStruct((NC, NS, ROWS_PER_TILE, DHALF),
                                    jnp.float32),
      mesh=mesh,
      scratch_types=[
          pltpu.VMEM((NCHUNK, CHUNK), jnp.int32),
          pltpu.VMEM((NCHUNK, CHUNK), jnp.int32),
          pltpu.VMEM((NCHUNK, CHUNK), jnp.float32),
          pltpu.VMEM((CHUNK, DHALF), jnp.float32),
          pltpu.VMEM_SHARED((N_NODES, DHALF), jnp.float32),
          pltpu.SemaphoreType.DMA,
      ],
      compiler_params=pltpu.CompilerParams(use_tc_tiling_on_sc=False),
  )
  def agg(x_hbm, col_hbm, row_hbm, val_hbm, z_hbm, out_hbm,
          col_v, row_v, val_v, gbuf, acc, sem):
    c = lax.axis_index("c")
    s = lax.axis_index("s")

    pltpu.sync_copy(z_hbm.at[s],
                    acc.at[pl.ds(s * ROWS_PER_TILE, ROWS_PER_TILE)])
    pltpu.sync_copy(col_hbm.at[s], col_v)
    pltpu.sync_copy(row_hbm.at[s], row_v)
    pltpu.sync_copy(val_hbm.at[s], val_v)
    plsc.subcore_barrier()

    xc = x_hbm.at[c]

    def chunk_body(k, carry):
      pltpu.async_copy(xc.at[col_v.at[k]], gbuf, sem).wait()

      def grp_body(g, c2):
        vv = val_v[k, pl.ds(g * LANES, LANES)]
        for e16 in range(LANES):
          v = vv[e16]
          e = g * LANES + e16
          for j in range(DHALF // LANES):
            sl = pl.ds(j * LANES, LANES)
            gbuf[e, sl] = gbuf[e, sl] * v
        return c2

      lax.fori_loop(0, CHUNK // LANES, grp_body, 0)
      pltpu.sync_copy(gbuf, acc.at[row_v.at[k]], add=True)
      return carry

    lax.fori_loop(0, NCHUNK, chunk_body, 0)

    plsc.subcore_barrier()
    pltpu.sync_copy(acc.at[pl.ds(s * ROWS_PER_TILE, ROWS_PER_TILE)],
                    out_hbm.at[c, s])

  return agg(x2, col3, row3, val3, zeros)


def _mm_body(a_ref, w_ref, o_ref):
  o_ref[...] = jnp.dot(a_ref[...], w_ref[...],
                       preferred_element_type=jnp.float32)


def _tc_matmul(a, w):
  bm = 1000
  return pl.pallas_call(
      _mm_body,
      grid=(N_NODES // bm,),
      in_specs=[
          pl.BlockSpec((bm, D_FEAT), lambda i: (i, 0)),
          pl.BlockSpec((D_FEAT, OUT_DIM), lambda i: (0, 0)),
      ],
      out_specs=pl.BlockSpec((bm, OUT_DIM), lambda i: (i, 0)),
      out_shape=jax.ShapeDtypeStruct((N_NODES, OUT_DIM), jnp.float32),
  )(a, w)


def kernel(x, adj_row, adj_col, adj_values, kernel):
  x2 = x.reshape(N_NODES, NC, DHALF).transpose(1, 0, 2)
  col3 = adj_col.reshape(NS, NCHUNK, CHUNK)
  row3 = adj_row.reshape(NS, NCHUNK, CHUNK)
  val3 = adj_values.reshape(NS, NCHUNK, CHUNK)
  zeros = jnp.zeros((NS, ROWS_PER_TILE, DHALF), jnp.float32)
  parts = _sc_aggregate(x2, col3, row3, val3, zeros)
  aggregated = parts.transpose(1, 2, 0, 3).reshape(N_NODES, D_FEAT)
  return _tc_matmul(aggregated, kernel)


# speedup vs baseline: 3.9905x; 1.3971x over previous
import functools

import jax
import jax.numpy as jnp
from jax import lax
from jax.experimental import pallas as pl
from jax.experimental.pallas import tpu as pltpu
from jax.experimental.pallas import tpu_sc as plsc

N_NODES = 10000
N_EDGES = 320000
D_FEAT = 128
OUT_DIM = 256

NC = 2
NS = 16
DHALF = D_FEAT // NC
E_PER_S = N_EDGES // NS
CHUNK = 80
NCHUNK = E_PER_S // CHUNK
ROWS_PER_TILE = N_NODES // NS
LANES = 16


def _sc_aggregate(x2, col3, row3, val3, zeros):
  mesh = plsc.VectorSubcoreMesh(core_axis_name="c", subcore_axis_name="s")

  @functools.partial(
      pl.kernel,
      out_type=jax.ShapeDtypeStruct((NC, NS, ROWS_PER_TILE, DHALF),
                                    jnp.float32),
      mesh=mesh,
      scratch_types=[
          pltpu.VMEM((NCHUNK, CHUNK), jnp.int32),
          pltpu.VMEM((NCHUNK, CHUNK), jnp.int32),
          pltpu.VMEM((NCHUNK, CHUNK), jnp.float32),
          pltpu.VMEM((2, CHUNK, DHALF), jnp.float32),
          pltpu.VMEM_SHARED((N_NODES, DHALF), jnp.float32),
          pltpu.SemaphoreType.DMA((2,)),
      ],
      compiler_params=pltpu.CompilerParams(use_tc_tiling_on_sc=False),
  )
  def agg(x_hbm, col_hbm, row_hbm, val_hbm, z_hbm, out_hbm,
          col_v, row_v, val_v, gbuf, acc, sem):
    c = lax.axis_index("c")
    s = lax.axis_index("s")

    pltpu.sync_copy(z_hbm.at[s],
                    acc.at[pl.ds(s * ROWS_PER_TILE, ROWS_PER_TILE)])
    pltpu.sync_copy(col_hbm.at[s], col_v)
    pltpu.sync_copy(row_hbm.at[s], row_v)
    pltpu.sync_copy(val_hbm.at[s], val_v)
    plsc.subcore_barrier()

    xc = x_hbm.at[c]

    def scale_chunk(k, b):
      def grp_body(g, c2):
        vv = val_v[k, pl.ds(g * LANES, LANES)]
        for e16 in range(LANES):
          v = vv[e16]
          e = g * LANES + e16
          for j in range(DHALF // LANES):
            sl = pl.ds(j * LANES, LANES)
            gbuf[b, e, sl] = gbuf[b, e, sl] * v
        return c2

      lax.fori_loop(0, CHUNK // LANES, grp_body, 0)

    pltpu.async_copy(xc.at[col_v.at[0]], gbuf.at[0], sem.at[0])

    def chunk_body(k, carry):
      b = lax.rem(k, 2)
      pltpu.make_async_copy(xc.at[col_v.at[k]], gbuf.at[b], sem.at[b]).wait()
      pltpu.async_copy(xc.at[col_v.at[k + 1]], gbuf.at[1 - b], sem.at[1 - b])
      scale_chunk(k, b)
      pltpu.sync_copy(gbuf.at[b], acc.at[row_v.at[k]], add=True)
      return carry

    lax.fori_loop(0, NCHUNK - 1, chunk_body, 0)

    k_last = NCHUNK - 1
    b_last = lax.rem(k_last, 2)
    pltpu.make_async_copy(xc.at[col_v.at[k_last]], gbuf.at[b_last],
                          sem.at[b_last]).wait()
    scale_chunk(k_last, b_last)
    pltpu.sync_copy(gbuf.at[b_last], acc.at[row_v.at[k_last]], add=True)

    plsc.subcore_barrier()
    pltpu.sync_copy(acc.at[pl.ds(s * ROWS_PER_TILE, ROWS_PER_TILE)],
                    out_hbm.at[c, s])

  return agg(x2, col3, row3, val3, zeros)


def _mm_body(a_ref, w_ref, o_ref):
  o_ref[...] = jnp.dot(a_ref[...], w_ref[...],
                       preferred_element_type=jnp.float32)


def _tc_matmul(a, w):
  bm = 1000
  return pl.pallas_call(
      _mm_body,
      grid=(N_NODES // bm,),
      in_specs=[
          pl.BlockSpec((bm, D_FEAT), lambda i: (i, 0)),
          pl.BlockSpec((D_FEAT, OUT_DIM), lambda i: (0, 0)),
      ],
      out_specs=pl.BlockSpec((bm, OUT_DIM), lambda i: (i, 0)),
      out_shape=jax.ShapeDtypeStruct((N_NODES, OUT_DIM), jnp.float32),
  )(a, w)


def kernel(x, adj_row, adj_col, adj_values, kernel):
  x2 = x.reshape(N_NODES, NC, DHALF).transpose(1, 0, 2)
  col3 = adj_col.reshape(NS, NCHUNK, CHUNK)
  row3 = adj_row.reshape(NS, NCHUNK, CHUNK)
  val3 = adj_values.reshape(NS, NCHUNK, CHUNK)
  zeros = jnp.zeros((NS, ROWS_PER_TILE, DHALF), jnp.float32)
  parts = _sc_aggregate(x2, col3, row3, val3, zeros)
  aggregated = parts.transpose(1, 2, 0, 3).reshape(N_NODES, D_FEAT)
  return _tc_matmul(aggregated, kernel)
